# Initial kernel scaffold; baseline (speedup 1.0000x reference)
import jax
import jax.numpy as jnp
from jax.experimental import pallas as pl


def kernel(rpn_probs, rpn_deltas, anchors):
    raise NotImplementedError("write your pallas kernel here")


# speedup vs baseline: 184.7310x; 184.7310x over previous
import functools

import jax
import jax.numpy as jnp
from jax import lax
from jax.experimental import pallas as pl
from jax.experimental.pallas import tpu as pltpu
from jax.experimental.pallas import tpu_sc as plsc

_STD = (0.1, 0.1, 0.2, 0.2)
_PROPOSAL_COUNT = 2000
_THR = 0.7

_B, _N = 2, 5000
_NP = 5120
_NB = _NP // 128
_NW = 32
_ROWS1 = _NP // _NW
_OUTP = 2048
_ROWS2 = _OUTP // _NW
_SENTINEL = 1 << 20
_CH = 1024
_TB = 256
_NT = _NP // _TB


_RW = 512


def _decode_rank_kernel(srow_ref, scol_ref, d_ref, a_ref, boxes_ref, rank_ref):
    a0 = a_ref[0, 0]
    a1 = a_ref[0, 1]
    a2 = a_ref[0, 2]
    a3 = a_ref[0, 3]
    d0 = d_ref[0, 0] * _STD[0]
    d1 = d_ref[0, 1] * _STD[1]
    d2 = d_ref[0, 2] * _STD[2]
    d3 = d_ref[0, 3] * _STD[3]
    h = a2 - a0
    w = a3 - a1
    cy = a0 + 0.5 * h
    cx = a1 + 0.5 * w
    cy = cy + d0 * h
    cx = cx + d1 * w
    h = h * jnp.exp(d2)
    w = w * jnp.exp(d3)
    y1 = cy - 0.5 * h
    x1 = cx - 0.5 * w
    y2 = y1 + h
    x2 = x1 + w
    boxes_ref[0, 0] = jnp.clip(y1, 0.0, 1.0)
    boxes_ref[0, 1] = jnp.clip(x1, 0.0, 1.0)
    boxes_ref[0, 2] = jnp.clip(y2, 0.0, 1.0)
    boxes_ref[0, 3] = jnp.clip(x2, 0.0, 1.0)

    scol = scol_ref[0]
    jcol = lax.broadcasted_iota(jnp.int32, (_NP, 1), 0)

    def chunk(ic, _):
        srow = srow_ref[0, pl.ds(0, 1), pl.ds(ic * _RW, _RW)]
        irow = lax.broadcasted_iota(jnp.int32, (1, _RW), 1) + ic * _RW
        beats = (scol > srow) | ((scol == srow) & (jcol < irow))
        rank = jnp.sum(beats.astype(jnp.float32), axis=0, keepdims=True)
        rank_ref[0, pl.ds(0, 1), pl.ds(ic * _RW, _RW)] = rank
        return 0

    lax.fori_loop(0, _NP // _RW, chunk, 0)


def _nms_kernel(rows_ref, cols_ref, keep_ref, pos_ref, m_ref, area_ref):
    m_ref[pl.ds(0, 4)] = jnp.full((4, _NP, 1), 2.0, jnp.float32)
    m_ref[4] = jnp.zeros((_NP, 1), jnp.float32)
    area_ref[...] = ((cols_ref[0, 2] - cols_ref[0, 0])
                     * (cols_ref[0, 3] - cols_ref[0, 1]))

    r2 = lax.broadcasted_iota(jnp.int32, (_TB, _TB), 0)
    c2 = lax.broadcasted_iota(jnp.int32, (_TB, _TB), 1)
    upper = (r2 < c2).astype(jnp.float32)
    eye = (r2 == c2).astype(jnp.float32)
    tri = (r2 <= c2).astype(jnp.float32)
    lane = lax.broadcasted_iota(jnp.int32, (1, _TB), 1)

    def block(b, base):
        y1b = rows_ref[0, 0, pl.ds(b, 1), :]
        x1b = rows_ref[0, 1, pl.ds(b, 1), :]
        y2b = rows_ref[0, 2, pl.ds(b, 1), :]
        x2b = rows_ref[0, 3, pl.ds(b, 1), :]
        area_b = (y2b - y1b) * (x2b - x1b)

        def sweep(jc, acc):
            rs = pl.ds(jc * _CH, _CH)
            yA = jnp.maximum(m_ref[0, rs, :], y1b)
            xA = jnp.maximum(m_ref[1, rs, :], x1b)
            yB = jnp.minimum(m_ref[2, rs, :], y2b)
            xB = jnp.minimum(m_ref[3, rs, :], x2b)
            inter = jnp.maximum(yB - yA, 0.0) * jnp.maximum(xB - xA, 0.0)
            union = m_ref[4, rs, :] + area_b - inter
            iou = inter / jnp.maximum(union, 1e-10)
            sup = (iou > _THR).astype(jnp.float32)
            return jnp.maximum(acc, jnp.max(sup, axis=0, keepdims=True))

        nchunks = (b * _TB + _CH - 1) // _CH
        supped = lax.fori_loop(0, nchunks, sweep,
                               jnp.zeros((1, _TB), jnp.float32))
        valid = (supped == 0.0) & (lane + b * _TB < _N)
        valid_f = valid.astype(jnp.float32)

        y1d = cols_ref[0, 0, pl.ds(b * _TB, _TB), :]
        x1d = cols_ref[0, 1, pl.ds(b * _TB, _TB), :]
        y2d = cols_ref[0, 2, pl.ds(b * _TB, _TB), :]
        x2d = cols_ref[0, 3, pl.ds(b * _TB, _TB), :]
        area_d = area_ref[pl.ds(b * _TB, _TB), :]
        yA2 = jnp.maximum(y1d, y1b)
        xA2 = jnp.maximum(x1d, x1b)
        yB2 = jnp.minimum(y2d, y2b)
        xB2 = jnp.minimum(x2d, x2b)
        inter2 = jnp.maximum(yB2 - yA2, 0.0) * jnp.maximum(xB2 - xA2, 0.0)
        union2 = area_d + area_b - inter2
        iou2 = inter2 / jnp.maximum(union2, 1e-10)
        S = (iou2 > _THR).astype(jnp.float32) * upper

        def cond(carry):
            return carry[1]

        def body(carry):
            keep, _ = carry
            cnt = lax.dot_general(keep, S, (((1,), (0,)), ((), ())),
                                  preferred_element_type=jnp.float32)
            keep_new = valid_f * (cnt == 0.0).astype(jnp.float32)
            changed = jnp.any(keep_new != keep)
            return keep_new, changed

        keep_b, _ = lax.while_loop(cond, body, (valid_f, True))

        keep_ref[0, pl.ds(b, 1), :] = keep_b
        kT = lax.dot_general(eye, keep_b, (((1,), (1,)), ((), ())),
                             preferred_element_type=jnp.float32)
        km = kT > 0.0
        m_ref[0, pl.ds(b * _TB, _TB), :] = jnp.where(km, y1d, 2.0)
        m_ref[1, pl.ds(b * _TB, _TB), :] = jnp.where(km, x1d, 2.0)
        m_ref[2, pl.ds(b * _TB, _TB), :] = jnp.where(km, y2d, 2.0)
        m_ref[3, pl.ds(b * _TB, _TB), :] = jnp.where(km, x2d, 2.0)
        m_ref[4, pl.ds(b * _TB, _TB), :] = jnp.where(km, area_d, 0.0)

        cum = lax.dot_general(keep_b, tri, (((1,), (0,)), ((), ())),
                              preferred_element_type=jnp.float32)
        pos = base + cum - 1.0
        pos_ref[0, pl.ds(b, 1), :] = jnp.where(
            keep_b > 0.0, pos, jnp.float32(_SENTINEL))
        return base + jnp.sum(keep_b)

    lax.fori_loop(0, _NT, block, jnp.float32(0.0))


def _make_sc_permute():
    mesh = plsc.VectorSubcoreMesh(core_axis_name="c", subcore_axis_name="s")

    def body(idx_hbm, val_hbm, out_hbm, idx_v, val_v, loc_v):
        wid = lax.axis_index("s") * 2 + lax.axis_index("c")
        lo = wid * _ROWS1
        for b in range(_B):
            pltpu.sync_copy(idx_hbm.at[pl.ds(b * _NP, _NP)], idx_v)
            pltpu.sync_copy(val_hbm.at[pl.ds(b * 4 * _NP, 4 * _NP)], val_v)

            def chunk(i, _):
                r16 = idx_v[pl.ds(i * 16, 16)]
                m = (r16 >= lo) & (r16 < lo + _ROWS1)
                rloc = r16 - lo
                for c in range(4):
                    v16 = val_v[pl.ds(c * _NP + i * 16, 16)]
                    c16 = jnp.full((16,), c, jnp.int32)
                    plsc.store_scatter(loc_v, [c16, rloc], v16, mask=m)
                return 0

            lax.fori_loop(0, _NP // 16, chunk, 0)
            for c in range(4):
                pltpu.sync_copy(
                    loc_v.at[c],
                    out_hbm.at[pl.ds(b * 4 * _NP + c * _NP + lo, _ROWS1)])

    return pl.kernel(
        body,
        out_type=jax.ShapeDtypeStruct((_B * 4 * _NP,), jnp.float32),
        mesh=mesh,
        compiler_params=pltpu.CompilerParams(
            use_tc_tiling_on_sc=False, needs_layout_passes=False),
        scratch_types=[
            pltpu.VMEM((_NP,), jnp.int32),
            pltpu.VMEM((4 * _NP,), jnp.float32),
            pltpu.VMEM((4, _ROWS1), jnp.float32),
        ],
    )


def _make_sc_compact():
    mesh = plsc.VectorSubcoreMesh(core_axis_name="c", subcore_axis_name="s")
    nrow = _ROWS2 * 4 // 16

    def body(idx_hbm, val_hbm, out_hbm, idx_v, val_v, loc_v):
        wid = lax.axis_index("s") * 2 + lax.axis_index("c")
        lo = wid * _ROWS2
        out_sz = _OUTP * 4
        for b in range(_B):
            pltpu.sync_copy(idx_hbm.at[pl.ds(b * _NP, _NP)], idx_v)
            pltpu.sync_copy(val_hbm.at[pl.ds(b * 4 * _NP, 4 * _NP)], val_v)
            for i in range(nrow):
                loc_v[i] = jnp.zeros((16,), jnp.float32)

            def chunk(i, _):
                r16 = idx_v[pl.ds(i * 16, 16)]
                m = (r16 >= lo) & (r16 < lo + _ROWS2)
                rloc = r16 - lo
                for c in range(4):
                    v16 = val_v[pl.ds(c * _NP + i * 16, 16)]
                    f = rloc * 4 + c
                    plsc.store_scatter(
                        loc_v, [lax.shift_right_logical(f, 4), f & 15],
                        v16, mask=m)
                return 0

            lax.fori_loop(0, _NP // 16, chunk, 0)
            for i in range(nrow):
                pltpu.sync_copy(
                    loc_v.at[i],
                    out_hbm.at[pl.ds(b * out_sz + lo * 4 + i * 16, 16)])

    return pl.kernel(
        body,
        out_type=jax.ShapeDtypeStruct((_B * _OUTP * 4,), jnp.float32),
        mesh=mesh,
        compiler_params=pltpu.CompilerParams(
            use_tc_tiling_on_sc=False, needs_layout_passes=False),
        scratch_types=[
            pltpu.VMEM((_NP,), jnp.int32),
            pltpu.VMEM((4 * _NP,), jnp.float32),
            pltpu.VMEM((nrow, 16), jnp.float32),
        ],
    )


def _tc1(srow, scol, d4, a4, interpret=False):
    f32 = jnp.float32
    return pl.pallas_call(
        _decode_rank_kernel,
        grid=(_B,),
        in_specs=[
            pl.BlockSpec((1, 1, _NP), lambda b: (b, 0, 0)),
            pl.BlockSpec((1, _NP, 1), lambda b: (b, 0, 0)),
            pl.BlockSpec((1, 4, _NB, 128), lambda b: (b, 0, 0, 0)),
            pl.BlockSpec((1, 4, _NB, 128), lambda b: (b, 0, 0, 0)),
        ],
        out_specs=[
            pl.BlockSpec((1, 4, _NB, 128), lambda b: (b, 0, 0, 0)),
            pl.BlockSpec((1, 1, _NP), lambda b: (b, 0, 0)),
        ],
        out_shape=[
            jax.ShapeDtypeStruct((_B, 4, _NB, 128), f32),
            jax.ShapeDtypeStruct((_B, 1, _NP), f32),
        ],
        interpret=interpret,
    )(srow, scol, d4, a4)


def _tc2(rows, cols, interpret=False):
    f32 = jnp.float32
    return pl.pallas_call(
        _nms_kernel,
        grid=(_B,),
        in_specs=[
            pl.BlockSpec((1, 4, _NT, _TB), lambda b: (b, 0, 0, 0)),
            pl.BlockSpec((1, 4, _NP, 1), lambda b: (b, 0, 0, 0)),
        ],
        out_specs=[
            pl.BlockSpec((1, _NT, _TB), lambda b: (b, 0, 0)),
            pl.BlockSpec((1, _NT, _TB), lambda b: (b, 0, 0)),
        ],
        out_shape=[
            jax.ShapeDtypeStruct((_B, _NT, _TB), f32),
            jax.ShapeDtypeStruct((_B, _NT, _TB), f32),
        ],
        scratch_shapes=[pltpu.VMEM((5, _NP, 1), f32),
                        pltpu.VMEM((_NP, 1), f32)],
        interpret=interpret,
    )(rows, cols)


@jax.jit
def kernel(rpn_probs, rpn_deltas, anchors):
    scores = rpn_probs[:, :, 1]
    pad = _NP - _N
    scores_p = jnp.pad(scores, ((0, 0), (0, pad)), constant_values=-1.0)
    srow = scores_p.reshape(_B, 1, _NP)
    scol = scores_p.reshape(_B, _NP, 1)
    d_t = jnp.pad(rpn_deltas.transpose(0, 2, 1), ((0, 0), (0, 0), (0, pad)))
    a_t = jnp.pad(anchors.transpose(0, 2, 1), ((0, 0), (0, 0), (0, pad)))
    d4 = d_t.reshape(_B, 4, _NB, 128)
    a4 = a_t.reshape(_B, 4, _NB, 128)

    boxes, rank = _tc1(srow, scol, d4, a4)

    rank_i = rank.astype(jnp.int32).reshape(_B * _NP)
    boxes_flat = boxes.reshape(_B * 4 * _NP)

    sorted_flat = _make_sc_permute()(rank_i, boxes_flat)

    rows = sorted_flat.reshape(_B, 4, _NT, _TB)
    cols = sorted_flat.reshape(_B, 4, _NP, 1)

    keep, posf = _tc2(rows, cols)

    del keep
    pos_i = posf.astype(jnp.int32).reshape(_B * _NP)

    out_flat = _make_sc_compact()(pos_i, sorted_flat)
    return out_flat.reshape(_B, _OUTP, 4)[:, :_PROPOSAL_COUNT, :]


# speedup vs baseline: 207.0132x; 1.1206x over previous
import functools

import jax
import jax.numpy as jnp
from jax import lax
from jax.experimental import pallas as pl
from jax.experimental.pallas import tpu as pltpu
from jax.experimental.pallas import tpu_sc as plsc

_STD = (0.1, 0.1, 0.2, 0.2)
_PROPOSAL_COUNT = 2000
_THR = 0.7

_B, _N = 2, 5000
_NP = 5120
_NB = _NP // 128
_NW = 32
_ROWS1 = _NP // _NW
_OUTP = 2048
_ROWS2 = _OUTP // _NW
_SENTINEL = 1 << 20
_CH = 1024
_TB = 512
_NT = _NP // _TB


_RW = 512


def _decode_rank_kernel(srow_ref, scol_ref, d_ref, a_ref, boxes_ref, rank_ref):
    a0 = a_ref[0, 0]
    a1 = a_ref[0, 1]
    a2 = a_ref[0, 2]
    a3 = a_ref[0, 3]
    d0 = d_ref[0, 0] * _STD[0]
    d1 = d_ref[0, 1] * _STD[1]
    d2 = d_ref[0, 2] * _STD[2]
    d3 = d_ref[0, 3] * _STD[3]
    h = a2 - a0
    w = a3 - a1
    cy = a0 + 0.5 * h
    cx = a1 + 0.5 * w
    cy = cy + d0 * h
    cx = cx + d1 * w
    h = h * jnp.exp(d2)
    w = w * jnp.exp(d3)
    y1 = cy - 0.5 * h
    x1 = cx - 0.5 * w
    y2 = y1 + h
    x2 = x1 + w
    boxes_ref[0, 0] = jnp.clip(y1, 0.0, 1.0)
    boxes_ref[0, 1] = jnp.clip(x1, 0.0, 1.0)
    boxes_ref[0, 2] = jnp.clip(y2, 0.0, 1.0)
    boxes_ref[0, 3] = jnp.clip(x2, 0.0, 1.0)

    scol = scol_ref[0]
    jcol = lax.broadcasted_iota(jnp.int32, (_NP, 1), 0)

    def chunk(ic, _):
        srow = srow_ref[0, pl.ds(0, 1), pl.ds(ic * _RW, _RW)]
        irow = lax.broadcasted_iota(jnp.int32, (1, _RW), 1) + ic * _RW
        beats = (scol > srow) | ((scol == srow) & (jcol < irow))
        rank = jnp.sum(beats.astype(jnp.float32), axis=0, keepdims=True)
        rank_ref[0, pl.ds(0, 1), pl.ds(ic * _RW, _RW)] = rank
        return 0

    lax.fori_loop(0, _NP // _RW, chunk, 0)


def _nms_kernel(rows_ref, cols_ref, keep_ref, pos_ref, m_ref, area_ref):
    m_ref[pl.ds(0, 4)] = jnp.full((4, _NP, 1), 2.0, jnp.float32)
    m_ref[4] = jnp.zeros((_NP, 1), jnp.float32)
    area_ref[...] = ((cols_ref[0, 2] - cols_ref[0, 0])
                     * (cols_ref[0, 3] - cols_ref[0, 1]))

    r2 = lax.broadcasted_iota(jnp.int32, (_TB, _TB), 0)
    c2 = lax.broadcasted_iota(jnp.int32, (_TB, _TB), 1)
    upper = (r2 < c2).astype(jnp.float32)
    eye = (r2 == c2).astype(jnp.float32)
    tri = (r2 <= c2).astype(jnp.float32)
    lane = lax.broadcasted_iota(jnp.int32, (1, _TB), 1)

    def block(b, base):
        y1b = rows_ref[0, 0, pl.ds(b, 1), :]
        x1b = rows_ref[0, 1, pl.ds(b, 1), :]
        y2b = rows_ref[0, 2, pl.ds(b, 1), :]
        x2b = rows_ref[0, 3, pl.ds(b, 1), :]
        area_b = (y2b - y1b) * (x2b - x1b)

        def sweep(jc, acc):
            rs = pl.ds(jc * _CH, _CH)
            yA = jnp.maximum(m_ref[0, rs, :], y1b)
            xA = jnp.maximum(m_ref[1, rs, :], x1b)
            yB = jnp.minimum(m_ref[2, rs, :], y2b)
            xB = jnp.minimum(m_ref[3, rs, :], x2b)
            inter = jnp.maximum(yB - yA, 0.0) * jnp.maximum(xB - xA, 0.0)
            union = m_ref[4, rs, :] + area_b - inter
            iou = inter / jnp.maximum(union, 1e-10)
            sup = (iou > _THR).astype(jnp.float32)
            return jnp.maximum(acc, jnp.max(sup, axis=0, keepdims=True))

        nchunks = (b * _TB + _CH - 1) // _CH
        supped = lax.fori_loop(0, nchunks, sweep,
                               jnp.zeros((1, _TB), jnp.float32))
        valid = (supped == 0.0) & (lane + b * _TB < _N)
        valid_f = valid.astype(jnp.float32)

        y1d = cols_ref[0, 0, pl.ds(b * _TB, _TB), :]
        x1d = cols_ref[0, 1, pl.ds(b * _TB, _TB), :]
        y2d = cols_ref[0, 2, pl.ds(b * _TB, _TB), :]
        x2d = cols_ref[0, 3, pl.ds(b * _TB, _TB), :]
        area_d = area_ref[pl.ds(b * _TB, _TB), :]
        yA2 = jnp.maximum(y1d, y1b)
        xA2 = jnp.maximum(x1d, x1b)
        yB2 = jnp.minimum(y2d, y2b)
        xB2 = jnp.minimum(x2d, x2b)
        inter2 = jnp.maximum(yB2 - yA2, 0.0) * jnp.maximum(xB2 - xA2, 0.0)
        union2 = area_d + area_b - inter2
        iou2 = inter2 / jnp.maximum(union2, 1e-10)
        S = (iou2 > _THR).astype(jnp.float32) * upper

        def cond(carry):
            return carry[1]

        def body(carry):
            keep, _ = carry
            cnt = lax.dot_general(keep, S, (((1,), (0,)), ((), ())),
                                  preferred_element_type=jnp.float32)
            keep_new = valid_f * (cnt == 0.0).astype(jnp.float32)
            changed = jnp.any(keep_new != keep)
            return keep_new, changed

        keep_b, _ = lax.while_loop(cond, body, (valid_f, True))

        keep_ref[0, pl.ds(b, 1), :] = keep_b
        kT = lax.dot_general(eye, keep_b, (((1,), (1,)), ((), ())),
                             preferred_element_type=jnp.float32)
        km = kT > 0.0
        m_ref[0, pl.ds(b * _TB, _TB), :] = jnp.where(km, y1d, 2.0)
        m_ref[1, pl.ds(b * _TB, _TB), :] = jnp.where(km, x1d, 2.0)
        m_ref[2, pl.ds(b * _TB, _TB), :] = jnp.where(km, y2d, 2.0)
        m_ref[3, pl.ds(b * _TB, _TB), :] = jnp.where(km, x2d, 2.0)
        m_ref[4, pl.ds(b * _TB, _TB), :] = jnp.where(km, area_d, 0.0)

        cum = lax.dot_general(keep_b, tri, (((1,), (0,)), ((), ())),
                              preferred_element_type=jnp.float32)
        pos = base + cum - 1.0
        pos_ref[0, pl.ds(b, 1), :] = jnp.where(
            keep_b > 0.0, pos, jnp.float32(_SENTINEL))
        return base + jnp.sum(keep_b)

    lax.fori_loop(0, _NT, block, jnp.float32(0.0))


def _make_sc_permute():
    mesh = plsc.VectorSubcoreMesh(core_axis_name="c", subcore_axis_name="s")

    def body(idx_hbm, val_hbm, out_hbm, idx_v, val_v, loc_v):
        wid = lax.axis_index("s") * 2 + lax.axis_index("c")
        lo = wid * _ROWS1
        for b in range(_B):
            pltpu.sync_copy(idx_hbm.at[pl.ds(b * _NP, _NP)], idx_v)
            pltpu.sync_copy(val_hbm.at[pl.ds(b * 4 * _NP, 4 * _NP)], val_v)

            def chunk(i, _):
                r16 = idx_v[pl.ds(i * 16, 16)]
                m = (r16 >= lo) & (r16 < lo + _ROWS1)
                rloc = r16 - lo
                for c in range(4):
                    v16 = val_v[pl.ds(c * _NP + i * 16, 16)]
                    c16 = jnp.full((16,), c, jnp.int32)
                    plsc.store_scatter(loc_v, [c16, rloc], v16, mask=m)
                return 0

            lax.fori_loop(0, _NP // 16, chunk, 0)
            for c in range(4):
                pltpu.sync_copy(
                    loc_v.at[c],
                    out_hbm.at[pl.ds(b * 4 * _NP + c * _NP + lo, _ROWS1)])

    return pl.kernel(
        body,
        out_type=jax.ShapeDtypeStruct((_B * 4 * _NP,), jnp.float32),
        mesh=mesh,
        compiler_params=pltpu.CompilerParams(
            use_tc_tiling_on_sc=False, needs_layout_passes=False),
        scratch_types=[
            pltpu.VMEM((_NP,), jnp.int32),
            pltpu.VMEM((4 * _NP,), jnp.float32),
            pltpu.VMEM((4, _ROWS1), jnp.float32),
        ],
    )


def _make_sc_compact():
    mesh = plsc.VectorSubcoreMesh(core_axis_name="c", subcore_axis_name="s")
    nrow = _ROWS2 * 4 // 16

    def body(idx_hbm, val_hbm, out_hbm, idx_v, val_v, loc_v):
        wid = lax.axis_index("s") * 2 + lax.axis_index("c")
        lo = wid * _ROWS2
        out_sz = _OUTP * 4
        for b in range(_B):
            pltpu.sync_copy(idx_hbm.at[pl.ds(b * _NP, _NP)], idx_v)
            pltpu.sync_copy(val_hbm.at[pl.ds(b * 4 * _NP, 4 * _NP)], val_v)
            for i in range(nrow):
                loc_v[i] = jnp.zeros((16,), jnp.float32)

            def chunk(i, _):
                r16 = idx_v[pl.ds(i * 16, 16)]
                m = (r16 >= lo) & (r16 < lo + _ROWS2)
                rloc = r16 - lo
                for c in range(4):
                    v16 = val_v[pl.ds(c * _NP + i * 16, 16)]
                    f = rloc * 4 + c
                    plsc.store_scatter(
                        loc_v, [lax.shift_right_logical(f, 4), f & 15],
                        v16, mask=m)
                return 0

            lax.fori_loop(0, _NP // 16, chunk, 0)
            for i in range(nrow):
                pltpu.sync_copy(
                    loc_v.at[i],
                    out_hbm.at[pl.ds(b * out_sz + lo * 4 + i * 16, 16)])

    return pl.kernel(
        body,
        out_type=jax.ShapeDtypeStruct((_B * _OUTP * 4,), jnp.float32),
        mesh=mesh,
        compiler_params=pltpu.CompilerParams(
            use_tc_tiling_on_sc=False, needs_layout_passes=False),
        scratch_types=[
            pltpu.VMEM((_NP,), jnp.int32),
            pltpu.VMEM((4 * _NP,), jnp.float32),
            pltpu.VMEM((nrow, 16), jnp.float32),
        ],
    )


def _tc1(srow, scol, d4, a4, interpret=False):
    f32 = jnp.float32
    return pl.pallas_call(
        _decode_rank_kernel,
        grid=(_B,),
        in_specs=[
            pl.BlockSpec((1, 1, _NP), lambda b: (b, 0, 0)),
            pl.BlockSpec((1, _NP, 1), lambda b: (b, 0, 0)),
            pl.BlockSpec((1, 4, _NB, 128), lambda b: (b, 0, 0, 0)),
            pl.BlockSpec((1, 4, _NB, 128), lambda b: (b, 0, 0, 0)),
        ],
        out_specs=[
            pl.BlockSpec((1, 4, _NB, 128), lambda b: (b, 0, 0, 0)),
            pl.BlockSpec((1, 1, _NP), lambda b: (b, 0, 0)),
        ],
        out_shape=[
            jax.ShapeDtypeStruct((_B, 4, _NB, 128), f32),
            jax.ShapeDtypeStruct((_B, 1, _NP), f32),
        ],
        interpret=interpret,
    )(srow, scol, d4, a4)


def _tc2(rows, cols, interpret=False):
    f32 = jnp.float32
    return pl.pallas_call(
        _nms_kernel,
        grid=(_B,),
        in_specs=[
            pl.BlockSpec((1, 4, _NT, _TB), lambda b: (b, 0, 0, 0)),
            pl.BlockSpec((1, 4, _NP, 1), lambda b: (b, 0, 0, 0)),
        ],
        out_specs=[
            pl.BlockSpec((1, _NT, _TB), lambda b: (b, 0, 0)),
            pl.BlockSpec((1, _NT, _TB), lambda b: (b, 0, 0)),
        ],
        out_shape=[
            jax.ShapeDtypeStruct((_B, _NT, _TB), f32),
            jax.ShapeDtypeStruct((_B, _NT, _TB), f32),
        ],
        scratch_shapes=[pltpu.VMEM((5, _NP, 1), f32),
                        pltpu.VMEM((_NP, 1), f32)],
        interpret=interpret,
    )(rows, cols)


@jax.jit
def kernel(rpn_probs, rpn_deltas, anchors):
    scores = rpn_probs[:, :, 1]
    pad = _NP - _N
    scores_p = jnp.pad(scores, ((0, 0), (0, pad)), constant_values=-1.0)
    srow = scores_p.reshape(_B, 1, _NP)
    scol = scores_p.reshape(_B, _NP, 1)
    d_t = jnp.pad(rpn_deltas.transpose(0, 2, 1), ((0, 0), (0, 0), (0, pad)))
    a_t = jnp.pad(anchors.transpose(0, 2, 1), ((0, 0), (0, 0), (0, pad)))
    d4 = d_t.reshape(_B, 4, _NB, 128)
    a4 = a_t.reshape(_B, 4, _NB, 128)

    boxes, rank = _tc1(srow, scol, d4, a4)

    rank_i = rank.astype(jnp.int32).reshape(_B * _NP)
    boxes_flat = boxes.reshape(_B * 4 * _NP)

    sorted_flat = _make_sc_permute()(rank_i, boxes_flat)

    rows = sorted_flat.reshape(_B, 4, _NT, _TB)
    cols = sorted_flat.reshape(_B, 4, _NP, 1)

    keep, posf = _tc2(rows, cols)

    del keep
    pos_i = posf.astype(jnp.int32).reshape(_B * _NP)

    out_flat = _make_sc_compact()(pos_i, sorted_flat)
    return out_flat.reshape(_B, _OUTP, 4)[:, :_PROPOSAL_COUNT, :]


# speedup vs baseline: 217.9331x; 1.0528x over previous
import jax
import jax.numpy as jnp
from jax import lax
from jax.experimental import pallas as pl
from jax.experimental.pallas import tpu as pltpu
from jax.experimental.pallas import tpu_sc as plsc

_STD = (0.1, 0.1, 0.2, 0.2)
_PROPOSAL_COUNT = 2000
_THR = 0.7

_B, _N = 2, 5000
_NP = 5120
_NB = _NP // 128
_NW = 32
_ROWS1 = _NP // _NW
_OUTP = 2048
_ROWS2 = _OUTP // _NW
_SENTINEL = 1 << 20
_CH = 1024
_TB = 1024
_NT = _NP // _TB


_RW = 512


def _decode_rank_kernel(srow_ref, scol_ref, d_ref, a_ref, boxes_ref, rank_ref):
    a0 = a_ref[0, 0]
    a1 = a_ref[0, 1]
    a2 = a_ref[0, 2]
    a3 = a_ref[0, 3]
    d0 = d_ref[0, 0] * _STD[0]
    d1 = d_ref[0, 1] * _STD[1]
    d2 = d_ref[0, 2] * _STD[2]
    d3 = d_ref[0, 3] * _STD[3]
    h = a2 - a0
    w = a3 - a1
    cy = a0 + 0.5 * h
    cx = a1 + 0.5 * w
    cy = cy + d0 * h
    cx = cx + d1 * w
    h = h * jnp.exp(d2)
    w = w * jnp.exp(d3)
    y1 = cy - 0.5 * h
    x1 = cx - 0.5 * w
    y2 = y1 + h
    x2 = x1 + w
    boxes_ref[0, 0] = jnp.clip(y1, 0.0, 1.0)
    boxes_ref[0, 1] = jnp.clip(x1, 0.0, 1.0)
    boxes_ref[0, 2] = jnp.clip(y2, 0.0, 1.0)
    boxes_ref[0, 3] = jnp.clip(x2, 0.0, 1.0)

    scol = scol_ref[0]
    jcol = lax.broadcasted_iota(jnp.int32, (_NP, 1), 0)

    def chunk(ic, _):
        srow = srow_ref[0, pl.ds(0, 1), pl.ds(ic * _RW, _RW)]
        irow = lax.broadcasted_iota(jnp.int32, (1, _RW), 1) + ic * _RW
        beats = (scol > srow) | ((scol == srow) & (jcol < irow))
        rank = jnp.sum(beats.astype(jnp.float32), axis=0, keepdims=True)
        rank_ref[0, pl.ds(0, 1), pl.ds(ic * _RW, _RW)] = rank
        return 0

    lax.fori_loop(0, _NP // _RW, chunk, 0)


def _nms_kernel(rows_ref, cols_ref, keep_ref, pos_ref, m_ref, area_ref):
    m_ref[pl.ds(0, 4)] = jnp.full((4, _NP, 1), 2.0, jnp.float32)
    m_ref[4] = jnp.zeros((_NP, 1), jnp.float32)
    area_ref[...] = ((cols_ref[0, 2] - cols_ref[0, 0])
                     * (cols_ref[0, 3] - cols_ref[0, 1]))

    r2 = lax.broadcasted_iota(jnp.int32, (_TB, _TB), 0)
    c2 = lax.broadcasted_iota(jnp.int32, (_TB, _TB), 1)
    upper = (r2 < c2).astype(jnp.float32)
    eye = (r2 == c2).astype(jnp.float32)
    tri = (r2 <= c2).astype(jnp.float32)
    lane = lax.broadcasted_iota(jnp.int32, (1, _TB), 1)

    def block(b, base):
        y1b = rows_ref[0, 0, pl.ds(b, 1), :]
        x1b = rows_ref[0, 1, pl.ds(b, 1), :]
        y2b = rows_ref[0, 2, pl.ds(b, 1), :]
        x2b = rows_ref[0, 3, pl.ds(b, 1), :]
        area_b = (y2b - y1b) * (x2b - x1b)

        def sweep(jc, acc):
            rs = pl.ds(jc * _CH, _CH)
            yA = jnp.maximum(m_ref[0, rs, :], y1b)
            xA = jnp.maximum(m_ref[1, rs, :], x1b)
            yB = jnp.minimum(m_ref[2, rs, :], y2b)
            xB = jnp.minimum(m_ref[3, rs, :], x2b)
            inter = jnp.maximum(yB - yA, 0.0) * jnp.maximum(xB - xA, 0.0)
            union = m_ref[4, rs, :] + area_b - inter
            iou = inter / jnp.maximum(union, 1e-10)
            sup = (iou > _THR).astype(jnp.float32)
            return jnp.maximum(acc, jnp.max(sup, axis=0, keepdims=True))

        nchunks = (b * _TB + _CH - 1) // _CH
        supped = lax.fori_loop(0, nchunks, sweep,
                               jnp.zeros((1, _TB), jnp.float32))
        valid = (supped == 0.0) & (lane + b * _TB < _N)
        valid_f = valid.astype(jnp.float32)

        y1d = cols_ref[0, 0, pl.ds(b * _TB, _TB), :]
        x1d = cols_ref[0, 1, pl.ds(b * _TB, _TB), :]
        y2d = cols_ref[0, 2, pl.ds(b * _TB, _TB), :]
        x2d = cols_ref[0, 3, pl.ds(b * _TB, _TB), :]
        area_d = area_ref[pl.ds(b * _TB, _TB), :]
        yA2 = jnp.maximum(y1d, y1b)
        xA2 = jnp.maximum(x1d, x1b)
        yB2 = jnp.minimum(y2d, y2b)
        xB2 = jnp.minimum(x2d, x2b)
        inter2 = jnp.maximum(yB2 - yA2, 0.0) * jnp.maximum(xB2 - xA2, 0.0)
        union2 = area_d + area_b - inter2
        iou2 = inter2 / jnp.maximum(union2, 1e-10)
        S = (iou2 > _THR).astype(jnp.float32) * upper

        def cond(carry):
            return carry[1]

        def body(carry):
            keep, _ = carry
            cnt = lax.dot_general(keep, S, (((1,), (0,)), ((), ())),
                                  preferred_element_type=jnp.float32)
            keep_new = valid_f * (cnt == 0.0).astype(jnp.float32)
            changed = jnp.any(keep_new != keep)
            return keep_new, changed

        keep_b, _ = lax.while_loop(cond, body, (valid_f, True))

        keep_ref[0, pl.ds(b, 1), :] = keep_b
        kT = lax.dot_general(eye, keep_b, (((1,), (1,)), ((), ())),
                             preferred_element_type=jnp.float32)
        km = kT > 0.0
        m_ref[0, pl.ds(b * _TB, _TB), :] = jnp.where(km, y1d, 2.0)
        m_ref[1, pl.ds(b * _TB, _TB), :] = jnp.where(km, x1d, 2.0)
        m_ref[2, pl.ds(b * _TB, _TB), :] = jnp.where(km, y2d, 2.0)
        m_ref[3, pl.ds(b * _TB, _TB), :] = jnp.where(km, x2d, 2.0)
        m_ref[4, pl.ds(b * _TB, _TB), :] = jnp.where(km, area_d, 0.0)

        cum = lax.dot_general(keep_b, tri, (((1,), (0,)), ((), ())),
                              preferred_element_type=jnp.float32)
        pos = base + cum - 1.0
        pos_ref[0, pl.ds(b, 1), :] = jnp.where(
            keep_b > 0.0, pos, jnp.float32(_SENTINEL))
        return base + jnp.sum(keep_b)

    lax.fori_loop(0, _NT, block, jnp.float32(0.0))


def _make_sc_permute():
    mesh = plsc.VectorSubcoreMesh(core_axis_name="c", subcore_axis_name="s")

    def body(idx_hbm, val_hbm, out_hbm, idx_v, val_v, loc_v):
        wid = lax.axis_index("s") * 2 + lax.axis_index("c")
        lo = wid * _ROWS1
        for b in range(_B):
            pltpu.sync_copy(idx_hbm.at[pl.ds(b * _NP, _NP)], idx_v)
            pltpu.sync_copy(val_hbm.at[pl.ds(b * 4 * _NP, 4 * _NP)], val_v)

            def chunk(i, _):
                r16 = idx_v[pl.ds(i * 16, 16)]
                m = (r16 >= lo) & (r16 < lo + _ROWS1)
                rloc = r16 - lo
                for c in range(4):
                    v16 = val_v[pl.ds(c * _NP + i * 16, 16)]
                    c16 = jnp.full((16,), c, jnp.int32)
                    plsc.store_scatter(loc_v, [c16, rloc], v16, mask=m)
                return 0

            lax.fori_loop(0, _NP // 16, chunk, 0)
            for c in range(4):
                pltpu.sync_copy(
                    loc_v.at[c],
                    out_hbm.at[pl.ds(b * 4 * _NP + c * _NP + lo, _ROWS1)])

    return pl.kernel(
        body,
        out_type=jax.ShapeDtypeStruct((_B * 4 * _NP,), jnp.float32),
        mesh=mesh,
        compiler_params=pltpu.CompilerParams(
            use_tc_tiling_on_sc=False, needs_layout_passes=False),
        scratch_types=[
            pltpu.VMEM((_NP,), jnp.int32),
            pltpu.VMEM((4 * _NP,), jnp.float32),
            pltpu.VMEM((4, _ROWS1), jnp.float32),
        ],
    )


def _make_sc_compact():
    mesh = plsc.VectorSubcoreMesh(core_axis_name="c", subcore_axis_name="s")
    nrow = _ROWS2 * 4 // 16

    def body(idx_hbm, val_hbm, out_hbm, idx_v, val_v, loc_v):
        wid = lax.axis_index("s") * 2 + lax.axis_index("c")
        lo = wid * _ROWS2
        out_sz = _OUTP * 4
        for b in range(_B):
            pltpu.sync_copy(idx_hbm.at[pl.ds(b * _NP, _NP)], idx_v)
            pltpu.sync_copy(val_hbm.at[pl.ds(b * 4 * _NP, 4 * _NP)], val_v)
            for i in range(nrow):
                loc_v[i] = jnp.zeros((16,), jnp.float32)

            def chunk(i, _):
                r16 = idx_v[pl.ds(i * 16, 16)]
                m = (r16 >= lo) & (r16 < lo + _ROWS2)
                rloc = r16 - lo
                for c in range(4):
                    v16 = val_v[pl.ds(c * _NP + i * 16, 16)]
                    f = rloc * 4 + c
                    plsc.store_scatter(
                        loc_v, [lax.shift_right_logical(f, 4), f & 15],
                        v16, mask=m)
                return 0

            lax.fori_loop(0, _NP // 16, chunk, 0)
            for i in range(nrow):
                pltpu.sync_copy(
                    loc_v.at[i],
                    out_hbm.at[pl.ds(b * out_sz + lo * 4 + i * 16, 16)])

    return pl.kernel(
        body,
        out_type=jax.ShapeDtypeStruct((_B * _OUTP * 4,), jnp.float32),
        mesh=mesh,
        compiler_params=pltpu.CompilerParams(
            use_tc_tiling_on_sc=False, needs_layout_passes=False),
        scratch_types=[
            pltpu.VMEM((_NP,), jnp.int32),
            pltpu.VMEM((4 * _NP,), jnp.float32),
            pltpu.VMEM((nrow, 16), jnp.float32),
        ],
    )


def _tc1(srow, scol, d4, a4, interpret=False):
    f32 = jnp.float32
    return pl.pallas_call(
        _decode_rank_kernel,
        grid=(_B,),
        in_specs=[
            pl.BlockSpec((1, 1, _NP), lambda b: (b, 0, 0)),
            pl.BlockSpec((1, _NP, 1), lambda b: (b, 0, 0)),
            pl.BlockSpec((1, 4, _NB, 128), lambda b: (b, 0, 0, 0)),
            pl.BlockSpec((1, 4, _NB, 128), lambda b: (b, 0, 0, 0)),
        ],
        out_specs=[
            pl.BlockSpec((1, 4, _NB, 128), lambda b: (b, 0, 0, 0)),
            pl.BlockSpec((1, 1, _NP), lambda b: (b, 0, 0)),
        ],
        out_shape=[
            jax.ShapeDtypeStruct((_B, 4, _NB, 128), f32),
            jax.ShapeDtypeStruct((_B, 1, _NP), f32),
        ],
        interpret=interpret,
    )(srow, scol, d4, a4)


def _tc2(rows, cols, interpret=False):
    f32 = jnp.float32
    return pl.pallas_call(
        _nms_kernel,
        grid=(_B,),
        in_specs=[
            pl.BlockSpec((1, 4, _NT, _TB), lambda b: (b, 0, 0, 0)),
            pl.BlockSpec((1, 4, _NP, 1), lambda b: (b, 0, 0, 0)),
        ],
        out_specs=[
            pl.BlockSpec((1, _NT, _TB), lambda b: (b, 0, 0)),
            pl.BlockSpec((1, _NT, _TB), lambda b: (b, 0, 0)),
        ],
        out_shape=[
            jax.ShapeDtypeStruct((_B, _NT, _TB), f32),
            jax.ShapeDtypeStruct((_B, _NT, _TB), f32),
        ],
        scratch_shapes=[pltpu.VMEM((5, _NP, 1), f32),
                        pltpu.VMEM((_NP, 1), f32)],
        interpret=interpret,
    )(rows, cols)


@jax.jit
def kernel(rpn_probs, rpn_deltas, anchors):
    scores = rpn_probs[:, :, 1]
    pad = _NP - _N
    scores_p = jnp.pad(scores, ((0, 0), (0, pad)), constant_values=-1.0)
    srow = scores_p.reshape(_B, 1, _NP)
    scol = scores_p.reshape(_B, _NP, 1)
    d_t = jnp.pad(rpn_deltas.transpose(0, 2, 1), ((0, 0), (0, 0), (0, pad)))
    a_t = jnp.pad(anchors.transpose(0, 2, 1), ((0, 0), (0, 0), (0, pad)))
    d4 = d_t.reshape(_B, 4, _NB, 128)
    a4 = a_t.reshape(_B, 4, _NB, 128)

    boxes, rank = _tc1(srow, scol, d4, a4)

    rank_i = rank.astype(jnp.int32).reshape(_B * _NP)
    boxes_flat = boxes.reshape(_B * 4 * _NP)

    sorted_flat = _make_sc_permute()(rank_i, boxes_flat)

    rows = sorted_flat.reshape(_B, 4, _NT, _TB)
    cols = sorted_flat.reshape(_B, 4, _NP, 1)

    keep, posf = _tc2(rows, cols)

    del keep
    pos_i = posf.astype(jnp.int32).reshape(_B * _NP)

    out_flat = _make_sc_compact()(pos_i, sorted_flat)
    return out_flat.reshape(_B, _OUTP, 4)[:, :_PROPOSAL_COUNT, :]
